# Initial kernel scaffold; baseline (speedup 1.0000x reference)
import jax
import jax.numpy as jnp
from jax.experimental import pallas as pl


def kernel(x, edge_index, W1, b1, W2, b2):
    raise NotImplementedError("write your pallas kernel here")


# speedup vs baseline: 34.9389x; 34.9389x over previous
import functools

import jax
import jax.numpy as jnp
from jax import lax
from jax.experimental import pallas as pl
from jax.experimental.pallas import tpu as pltpu
from jax.experimental.pallas import tpu_sc as plsc

_N = 10000
_E = 320000
_D = 128
_H = 16
_C = 40

_NC = 2
_NS = 16
_NW = _NC * _NS
_CK = 128
_EPW = _E // _NW
_NCH = 80
_EPW_PAD = _NCH * _CK
_NP = 10240
_RPT = _NP // _NS

_mesh = plsc.VectorSubcoreMesh(core_axis_name="c", subcore_axis_name="s")


@functools.partial(
    pl.kernel,
    out_type=jax.ShapeDtypeStruct((_NC, _NP), jnp.float32),
    mesh=_mesh,
    scratch_types=[
        pltpu.VMEM((_NCH, _CK), jnp.int32),
        pltpu.VMEM((_CK,), jnp.float32),
        pltpu.VMEM_SHARED((_NP,), jnp.float32),
    ],
)
def _sc_degree(dst_hbm, zeros_hbm, ones_hbm, deg_out, dst_v, ones_v, acc_sh):
    c = lax.axis_index("c")
    s = lax.axis_index("s")
    wid = s * _NC + c
    pltpu.sync_copy(dst_hbm.at[wid], dst_v)
    pltpu.sync_copy(ones_hbm, ones_v)
    pltpu.sync_copy(zeros_hbm.at[pl.ds(s * _RPT, _RPT)],
                    acc_sh.at[pl.ds(s * _RPT, _RPT)])
    plsc.subcore_barrier()

    def body(j, carry):
        pltpu.sync_copy(ones_v, acc_sh.at[dst_v.at[j]], add=True)
        return carry

    lax.fori_loop(0, _NCH, body, 0)
    plsc.subcore_barrier()
    pltpu.sync_copy(acc_sh.at[pl.ds(s * _RPT, _RPT)],
                    deg_out.at[c].at[pl.ds(s * _RPT, _RPT)])


@functools.partial(
    pl.kernel,
    out_type=jax.ShapeDtypeStruct((_NC, _NP, _H), jnp.float32),
    mesh=_mesh,
    compiler_params=pltpu.CompilerParams(use_tc_tiling_on_sc=False),
    scratch_types=[
        pltpu.VMEM((_NCH, _CK), jnp.int32),
        pltpu.VMEM((_NCH, _CK), jnp.int32),
        pltpu.VMEM((2, _CK, _H), jnp.float32),
        pltpu.VMEM_SHARED((_NP, _H), jnp.float32),
        pltpu.SemaphoreType.DMA,
        pltpu.SemaphoreType.DMA,
    ],
)
def _sc_aggregate(p_hbm, src_hbm, dst_hbm, zeros_hbm, agg_out,
                  src_v, dst_v, rows_v, acc_sh, sem0, sem1):
    c = lax.axis_index("c")
    s = lax.axis_index("s")
    wid = s * _NC + c
    pltpu.sync_copy(src_hbm.at[wid], src_v)
    pltpu.sync_copy(dst_hbm.at[wid], dst_v)
    pltpu.sync_copy(zeros_hbm.at[pl.ds(s * _RPT, _RPT)],
                    acc_sh.at[pl.ds(s * _RPT, _RPT)])
    plsc.subcore_barrier()

    sems = (sem0, sem1)
    pltpu.async_copy(p_hbm.at[src_v.at[0]], rows_v.at[0], sem0)
    pltpu.async_copy(p_hbm.at[src_v.at[1]], rows_v.at[1], sem1)

    def body(jj, carry):
        for b in range(2):
            j = jj * 2 + b
            pltpu.make_async_copy(p_hbm.at[src_v.at[j]], rows_v.at[b],
                                  sems[b]).wait()
            pltpu.sync_copy(rows_v.at[b], acc_sh.at[dst_v.at[j]], add=True)

            @pl.when(j + 2 < _NCH)
            def _():
                pltpu.async_copy(p_hbm.at[src_v.at[j + 2]], rows_v.at[b],
                                 sems[b])
        return carry

    lax.fori_loop(0, _NCH // 2, body, 0)
    plsc.subcore_barrier()
    pltpu.sync_copy(acc_sh.at[pl.ds(s * _RPT, _RPT)],
                    agg_out.at[c].at[pl.ds(s * _RPT, _RPT)])


def _dis(degp_ref):
    return lax.rsqrt(degp_ref[0] + degp_ref[1] + 1.0)[:_N]


def _tc_mm1_body(degp_ref, x_ref, w1_ref, p1_ref):
    h = jnp.dot(x_ref[...], w1_ref[...], preferred_element_type=jnp.float32)
    p1_ref[...] = h * _dis(degp_ref)


def _tc_mid_body(degp_ref, a1p_ref, p1_ref, b1_ref, p2_ref):
    dis = _dis(degp_ref)
    total = a1p_ref[0, :_N] + a1p_ref[1, :_N] + p1_ref[...]
    o1 = jnp.maximum(total * dis + b1_ref[...], 0.0)
    p2_ref[...] = o1 * dis


def _tc_out_body(degp_ref, a2p_ref, p2_ref, w2_ref, b2_ref, out_ref):
    dis = _dis(degp_ref)
    total = a2p_ref[0, :_N] + a2p_ref[1, :_N] + p2_ref[...]
    out_ref[...] = jnp.dot(total * dis, w2_ref[...],
                           preferred_element_type=jnp.float32) + b2_ref[...]


_tc_mm1 = pl.pallas_call(
    _tc_mm1_body, out_shape=jax.ShapeDtypeStruct((_N, _H), jnp.float32))
_tc_mid = pl.pallas_call(
    _tc_mid_body, out_shape=jax.ShapeDtypeStruct((_N, _H), jnp.float32))
_tc_out = pl.pallas_call(
    _tc_out_body, out_shape=jax.ShapeDtypeStruct((_N, _C), jnp.float32))


def kernel(x, edge_index, W1, b1, W2, b2):
    src = edge_index[0].astype(jnp.int32)
    dst = edge_index[1].astype(jnp.int32)
    pad = _EPW_PAD - _EPW
    src_r = jnp.pad(src.reshape(_NW, _EPW), ((0, 0), (0, pad))
                    ).reshape(_NW, _NCH, _CK)
    dst_r = jnp.pad(dst.reshape(_NW, _EPW), ((0, 0), (0, pad)),
                    constant_values=_N).reshape(_NW, _NCH, _CK)
    zeros_nh = jnp.zeros((_NP, _H), jnp.float32)
    zeros_np = jnp.zeros((_NP,), jnp.float32)
    ones_ck = jnp.ones((_CK,), jnp.float32)

    degp = _sc_degree(dst_r, zeros_np, ones_ck).reshape(_NC, _NP, 1)
    p1 = _tc_mm1(degp, x, W1)
    a1p = _sc_aggregate(p1, src_r, dst_r, zeros_nh)
    p2 = _tc_mid(degp, a1p, p1, b1.reshape(1, _H))
    a2p = _sc_aggregate(p2, src_r, dst_r, zeros_nh)
    return _tc_out(degp, a2p, p2, W2, b2.reshape(1, _C))


# speedup vs baseline: 41.1016x; 1.1764x over previous
import functools

import jax
import jax.numpy as jnp
from jax import lax
from jax.experimental import pallas as pl
from jax.experimental.pallas import tpu as pltpu
from jax.experimental.pallas import tpu_sc as plsc

_N = 10000
_E = 320000
_D = 128
_H = 16
_C = 40

_NC = 2
_NS = 16
_NW = _NC * _NS
_CK = 128
_EPW = _E // _NW
_NCH = 80
_EPW_PAD = _NCH * _CK
_NP = 10240
_RPT = _NP // _NS

_mesh = plsc.VectorSubcoreMesh(core_axis_name="c", subcore_axis_name="s")


@functools.partial(
    pl.kernel,
    out_type=jax.ShapeDtypeStruct((_NC, _NP), jnp.float32),
    mesh=_mesh,
    scratch_types=[
        pltpu.VMEM((_NCH, _CK), jnp.int32),
        pltpu.VMEM((_CK,), jnp.float32),
        pltpu.VMEM_SHARED((_NP,), jnp.float32),
        pltpu.SemaphoreType.DMA,
        pltpu.SemaphoreType.DMA,
        pltpu.SemaphoreType.DMA,
        pltpu.SemaphoreType.DMA,
    ],
)
def _sc_degree(dst_hbm, zeros_hbm, ones_hbm, deg_out, dst_v, ones_v, acc_sh,
               sem0, sem1, sem2, sem3):
    c = lax.axis_index("c")
    s = lax.axis_index("s")
    wid = s * _NC + c
    pltpu.sync_copy(dst_hbm.at[wid], dst_v)
    pltpu.sync_copy(ones_hbm, ones_v)
    pltpu.sync_copy(zeros_hbm.at[pl.ds(s * _RPT, _RPT)],
                    acc_sh.at[pl.ds(s * _RPT, _RPT)])
    plsc.subcore_barrier()

    sems = (sem0, sem1, sem2, sem3)

    def body(jj, carry):
        for b in range(4):
            j = jj * 4 + b

            @pl.when(j >= 4)
            def _():
                pltpu.make_async_copy(ones_v, acc_sh.at[dst_v.at[j - 4]],
                                      sems[b]).wait()

            pltpu.async_copy(ones_v, acc_sh.at[dst_v.at[j]], sems[b],
                             add=True)
        return carry

    lax.fori_loop(0, _NCH // 4, body, 0)
    for j in range(_NCH - 4, _NCH):
        pltpu.make_async_copy(ones_v, acc_sh.at[dst_v.at[j]],
                              sems[j % 4]).wait()
    plsc.subcore_barrier()
    pltpu.sync_copy(acc_sh.at[pl.ds(s * _RPT, _RPT)],
                    deg_out.at[c].at[pl.ds(s * _RPT, _RPT)])


@functools.partial(
    pl.kernel,
    out_type=jax.ShapeDtypeStruct((_NC, _NP, _H), jnp.float32),
    mesh=_mesh,
    compiler_params=pltpu.CompilerParams(use_tc_tiling_on_sc=False),
    scratch_types=[
        pltpu.VMEM((_NCH, _CK), jnp.int32),
        pltpu.VMEM((_NCH, _CK), jnp.int32),
        pltpu.VMEM((4, _CK, _H), jnp.float32),
        pltpu.VMEM_SHARED((_NP, _H), jnp.float32),
        pltpu.SemaphoreType.DMA,
        pltpu.SemaphoreType.DMA,
        pltpu.SemaphoreType.DMA,
        pltpu.SemaphoreType.DMA,
    ],
)
def _sc_aggregate(p_hbm, src_hbm, dst_hbm, zeros_hbm, agg_out,
                  src_v, dst_v, rows_v, acc_sh, sem0, sem1, sem2, sem3):
    c = lax.axis_index("c")
    s = lax.axis_index("s")
    wid = s * _NC + c
    pltpu.sync_copy(src_hbm.at[wid], src_v)
    pltpu.sync_copy(dst_hbm.at[wid], dst_v)
    pltpu.sync_copy(zeros_hbm.at[pl.ds(s * _RPT, _RPT)],
                    acc_sh.at[pl.ds(s * _RPT, _RPT)])
    plsc.subcore_barrier()

    sems = (sem0, sem1, sem2, sem3)
    pltpu.async_copy(p_hbm.at[src_v.at[0]], rows_v.at[0], sems[0])
    pltpu.async_copy(p_hbm.at[src_v.at[1]], rows_v.at[1], sems[1])

    def body(jj, carry):
        for b in range(4):
            j = jj * 4 + b
            b2 = (b + 2) % 4
            pltpu.make_async_copy(p_hbm.at[src_v.at[j]], rows_v.at[b],
                                  sems[b]).wait()
            pltpu.async_copy(rows_v.at[b], acc_sh.at[dst_v.at[j]], sems[b],
                             add=True)

            @pl.when(j >= 2)
            def _():
                pltpu.make_async_copy(rows_v.at[b2],
                                      acc_sh.at[dst_v.at[j - 2]],
                                      sems[b2]).wait()

            @pl.when(j + 2 < _NCH)
            def _():
                pltpu.async_copy(p_hbm.at[src_v.at[j + 2]], rows_v.at[b2],
                                 sems[b2])
        return carry

    lax.fori_loop(0, _NCH // 4, body, 0)
    for j in (_NCH - 2, _NCH - 1):
        b = j % 4
        pltpu.make_async_copy(rows_v.at[b], acc_sh.at[dst_v.at[j]],
                              sems[b]).wait()
    plsc.subcore_barrier()
    pltpu.sync_copy(acc_sh.at[pl.ds(s * _RPT, _RPT)],
                    agg_out.at[c].at[pl.ds(s * _RPT, _RPT)])


def _dis(degp_ref):
    return lax.rsqrt(degp_ref[0] + degp_ref[1] + 1.0)[:_N]


def _tc_mm1_body(x_ref, w1_ref, h_ref):
    h_ref[...] = jnp.dot(x_ref[...], w1_ref[...],
                         preferred_element_type=jnp.float32)


def _tc_scale_body(degp_ref, h_ref, p1_ref):
    p1_ref[...] = h_ref[...] * _dis(degp_ref)


def _tc_mid_body(degp_ref, a1p_ref, p1_ref, b1_ref, p2_ref):
    dis = _dis(degp_ref)
    total = a1p_ref[0, :_N] + a1p_ref[1, :_N] + p1_ref[...]
    o1 = jnp.maximum(total * dis + b1_ref[...], 0.0)
    p2_ref[...] = o1 * dis


def _tc_out_body(degp_ref, a2p_ref, p2_ref, w2_ref, b2_ref, out_ref):
    dis = _dis(degp_ref)
    total = a2p_ref[0, :_N] + a2p_ref[1, :_N] + p2_ref[...]
    out_ref[...] = jnp.dot(total * dis, w2_ref[...],
                           preferred_element_type=jnp.float32) + b2_ref[...]


_tc_mm1 = pl.pallas_call(
    _tc_mm1_body, out_shape=jax.ShapeDtypeStruct((_N, _H), jnp.float32))
_tc_scale = pl.pallas_call(
    _tc_scale_body, out_shape=jax.ShapeDtypeStruct((_N, _H), jnp.float32))
_tc_mid = pl.pallas_call(
    _tc_mid_body, out_shape=jax.ShapeDtypeStruct((_N, _H), jnp.float32))
_tc_out = pl.pallas_call(
    _tc_out_body, out_shape=jax.ShapeDtypeStruct((_N, _C), jnp.float32))


def kernel(x, edge_index, W1, b1, W2, b2):
    src = edge_index[0].astype(jnp.int32)
    dst = edge_index[1].astype(jnp.int32)
    pad = _EPW_PAD - _EPW
    src_r = jnp.pad(src.reshape(_NW, _EPW), ((0, 0), (0, pad))
                    ).reshape(_NW, _NCH, _CK)
    dst_r = jnp.pad(dst.reshape(_NW, _EPW), ((0, 0), (0, pad)),
                    constant_values=_N).reshape(_NW, _NCH, _CK)
    zeros_nh = jnp.zeros((_NP, _H), jnp.float32)
    zeros_np = jnp.zeros((_NP,), jnp.float32)
    ones_ck = jnp.ones((_CK,), jnp.float32)

    h = _tc_mm1(x, W1)
    degp = _sc_degree(dst_r, zeros_np, ones_ck).reshape(_NC, _NP, 1)
    p1 = _tc_scale(degp, h)
    a1p = _sc_aggregate(p1, src_r, dst_r, zeros_nh)
    p2 = _tc_mid(degp, a1p, p1, b1.reshape(1, _H))
    a2p = _sc_aggregate(p2, src_r, dst_r, zeros_nh)
    return _tc_out(degp, a2p, p2, W2, b2.reshape(1, _C))
